# baseline (speedup 1.0000x reference)
import jax
import jax.numpy as jnp
from jax.experimental import pallas as pl
from jax.experimental.pallas import tpu as pltpu

N_BASE = 4
NUM_ROWS = 256
NA = 5
N = 16
T = 1024


def _step_kernel(m_ref, out_ref, alpha_ref):
    t = pl.program_id(0)

    @pl.when(t == 0)
    def _init():
        alpha_ref[...] = jnp.zeros((N, NUM_ROWS), jnp.float32)
        out_ref[0] = jnp.zeros((N, NUM_ROWS), jnp.float32)

    @pl.when(t > 0)
    def _step():
        alpha = alpha_ref[...]
        M = m_ref[0].reshape(N, NUM_ROWS, NA)
        a2 = alpha.reshape(N, N_BASE, 64)
        s = [M[:, :, 0] + alpha]
        for i in range(N_BASE):
            s.append(M[:, :, 1 + i] + jnp.repeat(a2[:, i, :], N_BASE, axis=-1))
        m01 = jnp.maximum(s[0], s[1])
        m23 = jnp.maximum(s[2], s[3])
        mx = jnp.maximum(jnp.maximum(m01, m23), s[4])
        acc = jnp.exp(s[0] - mx)
        for j in range(1, NA):
            acc = acc + jnp.exp(s[j] - mx)
        new = jnp.log(acc) + mx
        alpha_ref[...] = new
        out_ref[0] = new


@jax.jit
def kernel(scores):
    grid = (T + 1,)
    out = pl.pallas_call(
        _step_kernel,
        grid=grid,
        in_specs=[
            pl.BlockSpec((1, N, NUM_ROWS * NA),
                         lambda t: (jnp.maximum(t - 1, 0), 0, 0)),
        ],
        out_specs=pl.BlockSpec((1, N, NUM_ROWS), lambda t: (t, 0, 0)),
        out_shape=jax.ShapeDtypeStruct((T + 1, N, NUM_ROWS), jnp.float32),
        scratch_shapes=[pltpu.VMEM((N, NUM_ROWS), jnp.float32)],
    )(scores)
    return out


# speedup vs baseline: 7.7227x; 7.7227x over previous
import jax
import jax.numpy as jnp
from jax.experimental import pallas as pl
from jax.experimental.pallas import tpu as pltpu

N_BASE = 4
NUM_ROWS = 256
NA = 5
N = 16
T = 1024


def _step_kernel(m_ref, out_ref, alpha_ref):
    t = pl.program_id(0)

    @pl.when(t == 0)
    def _init():
        alpha_ref[...] = jnp.zeros((N, NUM_ROWS), jnp.float32)
        out_ref[0] = jnp.zeros((N, NUM_ROWS), jnp.float32)

    @pl.when(t > 0)
    def _step():
        alpha = alpha_ref[...]
        s = [m_ref[0, 0] + alpha]
        for i in range(N_BASE):
            g = jnp.repeat(alpha[:, 64 * i:64 * (i + 1)], N_BASE, axis=-1)
            s.append(m_ref[0, 1 + i] + g)
        m01 = jnp.maximum(s[0], s[1])
        m23 = jnp.maximum(s[2], s[3])
        mx = jnp.maximum(jnp.maximum(m01, m23), s[4])
        acc = jnp.exp(s[0] - mx)
        for j in range(1, NA):
            acc = acc + jnp.exp(s[j] - mx)
        new = jnp.log(acc) + mx
        alpha_ref[...] = new
        out_ref[0] = new


@jax.jit
def kernel(scores):
    mt = scores.reshape(T, N, NUM_ROWS, NA).transpose(0, 3, 1, 2)
    out = pl.pallas_call(
        _step_kernel,
        grid=(T + 1,),
        in_specs=[
            pl.BlockSpec((1, NA, N, NUM_ROWS),
                         lambda t: (jnp.maximum(t - 1, 0), 0, 0, 0)),
        ],
        out_specs=pl.BlockSpec((1, N, NUM_ROWS), lambda t: (t, 0, 0)),
        out_shape=jax.ShapeDtypeStruct((T + 1, N, NUM_ROWS), jnp.float32),
        scratch_shapes=[pltpu.VMEM((N, NUM_ROWS), jnp.float32)],
    )(mt)
    return out
